# baseline (speedup 1.0000x reference)
import functools

import jax
import jax.numpy as jnp
from jax import lax
from jax.experimental import pallas as pl
from jax.experimental.pallas import tpu as pltpu
from jax.experimental.pallas import tpu_sc as plsc


def _make_agg(N, D, E, with_cnt):
    info = plsc.get_sparse_core_info()
    NC, NS = info.num_cores, info.num_subcores
    NW = NC * NS
    EPW = E // NW
    K = 80
    NCH = EPW // K
    RPT = -(-N // (NS * 8)) * 8
    NP = RPT * NS

    NB = 4
    NCHP = -(-NCH // NB) * NB

    outs = [jax.ShapeDtypeStruct((NC, NP, D), jnp.float32)]
    scratch = (
        [pltpu.VMEM((K,), jnp.int32) for _ in range(NB)]
        + [pltpu.VMEM((K,), jnp.int32) for _ in range(NB)]
        + [pltpu.VMEM((K, D), jnp.float32) for _ in range(NB)]
        + [pltpu.VMEM_SHARED((NP, D), jnp.float32)]
        + [pltpu.SemaphoreType.DMA for _ in range(NB)]
        + [pltpu.SemaphoreType.DMA for _ in range(NB)]
        + [pltpu.SemaphoreType.DMA]
    )
    if with_cnt:
        outs.append(jax.ShapeDtypeStruct((NC * NP,), jnp.float32))
        scratch.append(pltpu.VMEM((K,), jnp.float32))
        scratch.append(pltpu.VMEM((RPT,), jnp.float32))
        scratch.append(pltpu.VMEM_SHARED((NP,), jnp.float32))

    mesh = plsc.VectorSubcoreMesh(core_axis_name="c", subcore_axis_name="s")

    @functools.partial(
        pl.kernel, mesh=mesh,
        out_type=tuple(outs) if len(outs) > 1 else outs[0],
        scratch_types=scratch)
    def agg(h_hbm, src_hbm, dst_hbm, *rest):
        if with_cnt:
            (out_hbm, cnt_hbm), rest = rest[:2], rest[2:]
        else:
            (out_hbm,), rest = rest[:1], rest[1:]
        sidx = rest[0:NB]
        didx = rest[NB:2 * NB]
        rows = rest[2 * NB:3 * NB]
        acc = rest[3 * NB]
        semg = rest[3 * NB + 1:4 * NB + 1]
        sems = rest[4 * NB + 1:5 * NB + 1]
        semi = rest[5 * NB + 1]
        if with_cnt:
            ones, zcol, cnt = rest[5 * NB + 2:]
        c = lax.axis_index("c")
        s = lax.axis_index("s")
        base = (s * NC + c) * EPW

        def idx_load(j, u):
            pltpu.make_async_copy(
                src_hbm.at[pl.ds(base + j * K, K)], sidx[u], semi).start()
            pltpu.make_async_copy(
                dst_hbm.at[pl.ds(base + j * K, K)], didx[u], semi).start()

        def idx_wait(u):
            pltpu.make_async_copy(
                src_hbm.at[pl.ds(base, K)], sidx[u], semi).wait()
            pltpu.make_async_copy(
                dst_hbm.at[pl.ds(base, K)], didx[u], semi).wait()

        def gather_start(u):
            pltpu.async_copy(h_hbm.at[sidx[u]], rows[u], semg[u])

        def gather_wait(u):
            pltpu.make_async_copy(h_hbm.at[sidx[u]], rows[u], semg[u]).wait()

        def scatter_start(u):
            pltpu.async_copy(rows[u], acc.at[didx[u]], sems[u], add=True)
            if with_cnt:
                pltpu.sync_copy(ones, cnt.at[didx[u]], add=True)

        def scatter_wait(u):
            pltpu.make_async_copy(rows[u], acc.at[didx[u]], sems[u]).wait()

        zero16 = jnp.zeros((16,), jnp.float32)

        def zr(r, carry):
            for cc in range(D // 16):
                rows[0][r, pl.ds(cc * 16, 16)] = zero16
            return carry

        lax.fori_loop(0, K, zr, 0)
        for t in range(-(-RPT // K)):
            sz = min(K, RPT - t * K)
            pltpu.sync_copy(rows[0].at[pl.ds(0, sz)],
                            acc.at[pl.ds(s * RPT + t * K, sz)])
        if with_cnt:
            ones16 = jnp.ones((16,), jnp.float32)
            for g in range(K // 16):
                ones[pl.ds(g * 16, 16)] = ones16

            def zc(i, carry):
                zcol[pl.ds(i * 16, 16)] = zero16
                return carry

            lax.fori_loop(0, RPT // 16, zc, 0)
            if RPT % 16:
                zcol[pl.ds(RPT - 16, 16)] = zero16
            pltpu.sync_copy(zcol, cnt.at[pl.ds(s * RPT, RPT)])
        plsc.subcore_barrier()

        padN = jnp.full((16,), N, jnp.int32)

        def body(i, carry):
            for u in range(NB):
                j = i * NB + u
                v = (u + 2) % NB

                @pl.when(j >= NB)
                def _():
                    scatter_wait(u)

                idx_load(jnp.minimum(j, NCH - 1), u)

                @pl.when(j >= 2)
                def _():
                    gather_wait(v)
                    scatter_start(v)

                idx_wait(u)
                if NCHP != NCH:
                    @pl.when(j >= NCH)
                    def _():
                        for g in range(K // 16):
                            didx[u][pl.ds(g * 16, 16)] = padN

                gather_start(u)
            return carry

        lax.fori_loop(0, NCHP // NB, body, 0)
        for u in (2, 3):
            gather_wait(u)
            scatter_start(u)
        for u in range(NB):
            scatter_wait(u)
        plsc.subcore_barrier()

        pltpu.sync_copy(acc.at[pl.ds(s * RPT, RPT)],
                        out_hbm.at[c, pl.ds(s * RPT, RPT)])
        if with_cnt:
            pltpu.sync_copy(cnt.at[pl.ds(s * RPT, RPT)], zcol)
            pltpu.sync_copy(zcol, cnt_hbm.at[pl.ds(c * NP + s * RPT, RPT)])

    return agg


def _dense(parts, cntparts_t, h, Wl, bl, Wr, a):
    N, D = h.shape
    NC = parts.shape[0]
    NW = cntparts_t.shape[1]
    R = 1000
    grid = N // R

    def body(p_ref, c_ref, h_ref, wl_ref, bl_ref, wr_ref, a_ref, o_ref):
        agg = p_ref[0] + p_ref[1]
        cnt = jnp.sum(c_ref[...], axis=1)
        mean = agg / jnp.maximum(cnt, 1.0)[:, None]
        y = (jnp.dot(mean, wl_ref[...], preferred_element_type=jnp.float32)
             + bl_ref[...][None, :]
             + jnp.dot(h_ref[...], wr_ref[...],
                       preferred_element_type=jnp.float32))
        av = a_ref[...][None, :]
        o_ref[...] = jnp.where(y >= 0, y, av * y)

    return pl.pallas_call(
        body,
        grid=(grid,),
        in_specs=[
            pl.BlockSpec((NC, R, D), lambda i: (0, i, 0)),
            pl.BlockSpec((R, NW), lambda i: (i, 0)),
            pl.BlockSpec((R, D), lambda i: (i, 0)),
            pl.BlockSpec((D, D), lambda i: (0, 0)),
            pl.BlockSpec((D,), lambda i: (0,)),
            pl.BlockSpec((D, D), lambda i: (0, 0)),
            pl.BlockSpec((D,), lambda i: (0,)),
        ],
        out_specs=pl.BlockSpec((R, D), lambda i: (i, 0)),
        out_shape=jax.ShapeDtypeStruct((N, D), jnp.float32),
    )(parts, cntparts_t, h, Wl, bl, Wr, a)


def kernel(x, edge_index, batch_size, Wl0, bl0, Wr0, a0,
           Wl1, bl1, Wr1, a1, Wl2, bl2, Wr2, a2):
    N, D = x.shape
    E = edge_index.shape[1]

    info = plsc.get_sparse_core_info()
    NC = info.num_cores
    NW = NC * info.num_subcores
    NP = -(-N // (info.num_subcores * 8)) * 8 * info.num_subcores
    src = edge_index[0].astype(jnp.int32)
    dst = edge_index[1].astype(jnp.int32)

    agg0 = _make_agg(N, D, E, with_cnt=True)
    agg = _make_agg(N, D, E, with_cnt=False)

    parts, cntflat = agg0(x, src, dst)
    cntparts_t = cntflat.reshape(NC, NP)[:, :N].T
    h1 = _dense(parts, cntparts_t, x, Wl0, bl0, Wr0, a0)
    parts = agg(h1, src, dst)
    h2 = _dense(parts, cntparts_t, h1, Wl1, bl1, Wr1, a1)
    parts = agg(h2, src, dst)
    h3 = _dense(parts, cntparts_t, h2, Wl2, bl2, Wr2, a2)
    return lax.dynamic_slice_in_dim(h3, batch_size - 1024, 1024, axis=0)


# speedup vs baseline: 1.0288x; 1.0288x over previous
import functools

import jax
import jax.numpy as jnp
from jax import lax
from jax.experimental import pallas as pl
from jax.experimental.pallas import tpu as pltpu
from jax.experimental.pallas import tpu_sc as plsc


def _make_agg(N, D, E, with_cnt):
    info = plsc.get_sparse_core_info()
    NC, NS = info.num_cores, info.num_subcores
    NW = NC * NS
    EPW = E // NW
    K = 80
    NCH = EPW // K
    RPT = -(-N // (NS * 8)) * 8
    NP = RPT * NS

    NB = 4
    LAG = 3
    NCHP = -(-NCH // NB) * NB

    outs = [jax.ShapeDtypeStruct((NC, NP, D), jnp.float32)]
    scratch = (
        [pltpu.VMEM((K,), jnp.int32) for _ in range(NB)]
        + [pltpu.VMEM((K,), jnp.int32) for _ in range(NB)]
        + [pltpu.VMEM((K, D), jnp.float32) for _ in range(NB)]
        + [pltpu.VMEM_SHARED((NP, D), jnp.float32)]
        + [pltpu.SemaphoreType.DMA for _ in range(NB)]
        + [pltpu.SemaphoreType.DMA for _ in range(NB)]
        + [pltpu.SemaphoreType.DMA]
    )
    if with_cnt:
        outs.append(jax.ShapeDtypeStruct((NC * NP,), jnp.float32))
        scratch.append(pltpu.VMEM((K,), jnp.float32))
        scratch.append(pltpu.VMEM((RPT,), jnp.float32))
        scratch.append(pltpu.VMEM_SHARED((NP,), jnp.float32))

    mesh = plsc.VectorSubcoreMesh(core_axis_name="c", subcore_axis_name="s")

    @functools.partial(
        pl.kernel, mesh=mesh,
        out_type=tuple(outs) if len(outs) > 1 else outs[0],
        scratch_types=scratch)
    def agg(h_hbm, src_hbm, dst_hbm, *rest):
        if with_cnt:
            (out_hbm, cnt_hbm), rest = rest[:2], rest[2:]
        else:
            (out_hbm,), rest = rest[:1], rest[1:]
        sidx = rest[0:NB]
        didx = rest[NB:2 * NB]
        rows = rest[2 * NB:3 * NB]
        acc = rest[3 * NB]
        semg = rest[3 * NB + 1:4 * NB + 1]
        sems = rest[4 * NB + 1:5 * NB + 1]
        semi = rest[5 * NB + 1]
        if with_cnt:
            ones, zcol, cnt = rest[5 * NB + 2:]
        c = lax.axis_index("c")
        s = lax.axis_index("s")
        base = (s * NC + c) * EPW

        def idx_load(j, u):
            pltpu.make_async_copy(
                src_hbm.at[pl.ds(base + j * K, K)], sidx[u], semi).start()
            pltpu.make_async_copy(
                dst_hbm.at[pl.ds(base + j * K, K)], didx[u], semi).start()

        def idx_wait(u):
            pltpu.make_async_copy(
                src_hbm.at[pl.ds(base, K)], sidx[u], semi).wait()
            pltpu.make_async_copy(
                dst_hbm.at[pl.ds(base, K)], didx[u], semi).wait()

        def gather_start(u):
            pltpu.async_copy(h_hbm.at[sidx[u]], rows[u], semg[u])

        def gather_wait(u):
            pltpu.make_async_copy(h_hbm.at[sidx[u]], rows[u], semg[u]).wait()

        def scatter_start(u):
            pltpu.async_copy(rows[u], acc.at[didx[u]], sems[u], add=True)
            if with_cnt:
                pltpu.sync_copy(ones, cnt.at[didx[u]], add=True)

        def scatter_wait(u):
            pltpu.make_async_copy(rows[u], acc.at[didx[u]], sems[u]).wait()

        zero16 = jnp.zeros((16,), jnp.float32)

        def zr(r, carry):
            for cc in range(D // 16):
                rows[0][r, pl.ds(cc * 16, 16)] = zero16
            return carry

        lax.fori_loop(0, K, zr, 0)
        for t in range(-(-RPT // K)):
            sz = min(K, RPT - t * K)
            pltpu.sync_copy(rows[0].at[pl.ds(0, sz)],
                            acc.at[pl.ds(s * RPT + t * K, sz)])
        if with_cnt:
            ones16 = jnp.ones((16,), jnp.float32)
            for g in range(K // 16):
                ones[pl.ds(g * 16, 16)] = ones16

            def zc(i, carry):
                zcol[pl.ds(i * 16, 16)] = zero16
                return carry

            lax.fori_loop(0, RPT // 16, zc, 0)
            if RPT % 16:
                zcol[pl.ds(RPT - 16, 16)] = zero16
            pltpu.sync_copy(zcol, cnt.at[pl.ds(s * RPT, RPT)])
        plsc.subcore_barrier()

        padN = jnp.full((16,), N, jnp.int32)

        def body(i, carry):
            for u in range(NB):
                j = i * NB + u
                v = (u + NB - LAG) % NB

                @pl.when(j >= NB)
                def _():
                    scatter_wait(u)

                idx_load(jnp.minimum(j, NCH - 1), u)

                @pl.when(j >= LAG)
                def _():
                    gather_wait(v)
                    scatter_start(v)

                idx_wait(u)
                if NCHP != NCH:
                    @pl.when(j >= NCH)
                    def _():
                        for g in range(K // 16):
                            didx[u][pl.ds(g * 16, 16)] = padN

                gather_start(u)
            return carry

        lax.fori_loop(0, NCHP // NB, body, 0)
        for t in range(LAG):
            u = (NCHP - LAG + t) % NB
            gather_wait(u)
            scatter_start(u)
        for u in range(NB):
            scatter_wait(u)
        plsc.subcore_barrier()

        pltpu.sync_copy(acc.at[pl.ds(s * RPT, RPT)],
                        out_hbm.at[c, pl.ds(s * RPT, RPT)])
        if with_cnt:
            pltpu.sync_copy(cnt.at[pl.ds(s * RPT, RPT)], zcol)
            pltpu.sync_copy(zcol, cnt_hbm.at[pl.ds(c * NP + s * RPT, RPT)])

    return agg


def _dense(parts, cntparts_t, h, Wl, bl, Wr, a):
    N, D = h.shape
    NC = parts.shape[0]
    NW = cntparts_t.shape[1]
    R = 1000
    grid = N // R

    def body(p_ref, c_ref, h_ref, wl_ref, bl_ref, wr_ref, a_ref, o_ref):
        agg = p_ref[0] + p_ref[1]
        cnt = jnp.sum(c_ref[...], axis=1)
        mean = agg / jnp.maximum(cnt, 1.0)[:, None]
        y = (jnp.dot(mean, wl_ref[...], preferred_element_type=jnp.float32)
             + bl_ref[...][None, :]
             + jnp.dot(h_ref[...], wr_ref[...],
                       preferred_element_type=jnp.float32))
        av = a_ref[...][None, :]
        o_ref[...] = jnp.where(y >= 0, y, av * y)

    return pl.pallas_call(
        body,
        grid=(grid,),
        in_specs=[
            pl.BlockSpec((NC, R, D), lambda i: (0, i, 0)),
            pl.BlockSpec((R, NW), lambda i: (i, 0)),
            pl.BlockSpec((R, D), lambda i: (i, 0)),
            pl.BlockSpec((D, D), lambda i: (0, 0)),
            pl.BlockSpec((D,), lambda i: (0,)),
            pl.BlockSpec((D, D), lambda i: (0, 0)),
            pl.BlockSpec((D,), lambda i: (0,)),
        ],
        out_specs=pl.BlockSpec((R, D), lambda i: (i, 0)),
        out_shape=jax.ShapeDtypeStruct((N, D), jnp.float32),
    )(parts, cntparts_t, h, Wl, bl, Wr, a)


def kernel(x, edge_index, batch_size, Wl0, bl0, Wr0, a0,
           Wl1, bl1, Wr1, a1, Wl2, bl2, Wr2, a2):
    N, D = x.shape
    E = edge_index.shape[1]

    info = plsc.get_sparse_core_info()
    NC = info.num_cores
    NW = NC * info.num_subcores
    NP = -(-N // (info.num_subcores * 8)) * 8 * info.num_subcores
    src = edge_index[0].astype(jnp.int32)
    dst = edge_index[1].astype(jnp.int32)

    agg0 = _make_agg(N, D, E, with_cnt=True)
    agg = _make_agg(N, D, E, with_cnt=False)

    parts, cntflat = agg0(x, src, dst)
    cntparts_t = cntflat.reshape(NC, NP)[:, :N].T
    h1 = _dense(parts, cntparts_t, x, Wl0, bl0, Wr0, a0)
    parts = agg(h1, src, dst)
    h2 = _dense(parts, cntparts_t, h1, Wl1, bl1, Wr1, a1)
    parts = agg(h2, src, dst)
    h3 = _dense(parts, cntparts_t, h2, Wl2, bl2, Wr2, a2)
    return lax.dynamic_slice_in_dim(h3, batch_size - 1024, 1024, axis=0)
